# baseline (speedup 1.0000x reference)
import functools

import jax
import jax.numpy as jnp
from jax import lax
from jax.experimental import pallas as pl
from jax.experimental.pallas import tpu as pltpu
from jax.experimental.pallas import tpu_sc as plsc

_NC = 2
_NS = 16
_NW = _NC * _NS
_CH = 128
_K = 16
_NF = 8


def _copy_body(x_ref, o_ref):
    o_ref[...] = x_ref[...]


def _copy(x, block_rows=1000):
    n, d = x.shape
    return pl.pallas_call(
        _copy_body,
        grid=(n // block_rows,),
        in_specs=[pl.BlockSpec((block_rows, d), lambda i: (i, 0))],
        out_specs=pl.BlockSpec((block_rows, d), lambda i: (i, 0)),
        out_shape=jax.ShapeDtypeStruct((n, d), jnp.float32),
    )(x)


def _gcn_body(p_ref, deg_ref, b_ref, w_ref, o_ref):
    agg = p_ref[0] + p_ref[1]
    deg = jnp.sum(deg_ref[...], axis=0)[:, None]
    agg = agg / jnp.maximum(deg, 1.0)
    o_ref[...] = jnp.maximum(
        jnp.dot(agg, w_ref[...], preferred_element_type=jnp.float32) + b_ref[...],
        0.0,
    )


def _gcn_dense(p, deg_all, w, b, block_rows=1024):
    npad, d = p.shape[1], p.shape[2]
    dout = w.shape[1]
    grid = (npad // block_rows,)
    return pl.pallas_call(
        _gcn_body,
        grid=grid,
        in_specs=[
            pl.BlockSpec((2, block_rows, d), lambda i: (0, i, 0)),
            pl.BlockSpec((_NW, block_rows), lambda i: (0, i)),
            pl.BlockSpec((1, dout), lambda i: (0, 0)),
            pl.BlockSpec((d, dout), lambda i: (0, 0)),
        ],
        out_specs=pl.BlockSpec((block_rows, dout), lambda i: (i, 0)),
        out_shape=jax.ShapeDtypeStruct((npad, dout), jnp.float32),
    )(p, deg_all, b, w)


def _make_heads_body(n):
    def _heads_body(q_ref, deg_ref, b1_ref, w1_ref, wpg_ref, wpd_ref, wv_ref,
                    bpg_ref, bpd_ref, bv_ref, pi_ref, v_ref):
        agg = q_ref[0, pl.ds(0, n), :] + q_ref[1, pl.ds(0, n), :]
        deg = jnp.sum(deg_ref[:, pl.ds(0, n)], axis=0)[:, None]
        agg = agg / jnp.maximum(deg, 1.0)
        h2 = jnp.maximum(
            jnp.dot(agg, w1_ref[...], preferred_element_type=jnp.float32) + b1_ref[...],
            0.0,
        )
        mn = jnp.mean(h2, axis=0, keepdims=True)
        pi_ref[pl.ds(0, n), :] = (
            jnp.dot(h2, wpg_ref[...], preferred_element_type=jnp.float32) + bpg_ref[...]
        )
        pi_ref[pl.ds(n, 1), :] = (
            jnp.dot(mn, wpd_ref[...], preferred_element_type=jnp.float32) + bpd_ref[...]
        )
        v_ref[...] = jnp.dot(mn, wv_ref[...], preferred_element_type=jnp.float32) + bv_ref[...]
    return _heads_body


def _heads(n, q, deg_all, b1, w1, wpg, wpd, wv, bpg, bpd, bv):
    return pl.pallas_call(
        _make_heads_body(n),
        out_shape=(
            jax.ShapeDtypeStruct((n + 1, 1), jnp.float32),
            jax.ShapeDtypeStruct((1, 1), jnp.float32),
        ),
    )(q, deg_all, b1, w1, wpg, wpd, wv, bpg, bpd, bv)


@functools.lru_cache(maxsize=None)
def _make_deg(npad, nchunks):
    nstage = nchunks // _K
    assert nchunks % _K == 0

    mesh = plsc.VectorSubcoreMesh(core_axis_name="c", subcore_axis_name="s")

    def body(dst_hbm, deg_hbm, dst_v, deg_l):
        cid = lax.axis_index("c")
        sid = lax.axis_index("s")
        wid = sid * _NC + cid

        zeros16 = jnp.zeros((16,), jnp.float32)
        ones16 = jnp.ones((16,), jnp.float32)

        def z_body(i, _):
            deg_l[pl.ds(i * 16, 16)] = zeros16
            return 0
        lax.fori_loop(0, npad // 16, z_body, 0)

        def s_body(s, _):
            pltpu.sync_copy(dst_hbm.at[wid, pl.ds(s * _K, _K)], dst_v)

            def c_body(j, _):
                for l in range(_CH // 16):
                    idxv = dst_v[j, pl.ds(l * 16, 16)]
                    plsc.addupdate_scatter(deg_l, [idxv], ones16)
                return 0
            lax.fori_loop(0, _K, c_body, 0)
            return 0
        lax.fori_loop(0, nstage, s_body, 0)

        pltpu.sync_copy(deg_l, deg_hbm.at[wid])

    return pl.kernel(
        body,
        out_type=jax.ShapeDtypeStruct((_NW, npad), jnp.float32),
        mesh=mesh,
        compiler_params=pltpu.CompilerParams(
            use_tc_tiling_on_sc=False, needs_layout_passes=False),
        scratch_types=[
            pltpu.VMEM((_K, _CH), jnp.int32),
            pltpu.VMEM((npad,), jnp.float32),
        ],
    )


@functools.lru_cache(maxsize=None)
def _make_mp(hwrows, npad, nchunks0, nchunks1, d):
    rows_per_tile = npad // _NS
    n_wb = rows_per_tile // _CH
    per_sid = nchunks0 + nchunks1
    assert rows_per_tile % _CH == 0
    assert nchunks0 % _K == 0 and nchunks1 % _K == 0

    mesh = plsc.VectorSubcoreMesh(core_axis_name="c", subcore_axis_name="s")

    def body(hw_hbm, src_hbm, dst_hbm, part_hbm,
             src_v, dst_v, rows_v, rows1_v, agg_sh, sem, g1, s0, s1):
        cid = lax.axis_index("c")
        sid = lax.axis_index("s")
        start = sid * per_sid + cid * nchunks0
        nstage = jnp.where(cid == 0, nchunks0 // _K, nchunks1 // _K)

        zeros16 = jnp.zeros((16,), jnp.float32)

        def z_body(i, _):
            def z_inner(k, _):
                rows_v[i, pl.ds(k * 16, 16)] = zeros16
                return 0
            lax.fori_loop(0, d // 16, z_inner, 0)
            return 0
        lax.fori_loop(0, _CH, z_body, 0)

        for k in range(n_wb):
            r0 = sid * rows_per_tile + k * _CH
            pltpu.sync_copy(rows_v, agg_sh.at[pl.ds(r0, _CH)])
        plsc.subcore_barrier()

        def s_body(st, _):
            r0 = start + st * _K
            pltpu.sync_copy(src_hbm.at[pl.ds(r0, _K)], src_v)
            pltpu.sync_copy(dst_hbm.at[pl.ds(r0, _K)], dst_v)

            rows = (rows_v, rows1_v)
            gsem = (sem, g1)
            ssem = (s0, s1)
            g = [None] * _K
            sc = [None] * _K
            g[0] = pltpu.async_copy(hw_hbm.at[src_v.at[0]], rows[0], gsem[0])
            for c in range(1, _K):
                b = c % 2
                g[c - 1].wait()
                if c >= 2:
                    sc[c - 2].wait()
                sc[c - 1] = pltpu.async_copy(
                    rows[1 - b], agg_sh.at[dst_v.at[c - 1]], ssem[1 - b], add=True)
                g[c] = pltpu.async_copy(hw_hbm.at[src_v.at[c]], rows[b], gsem[b])
            g[_K - 1].wait()
            sc[_K - 2].wait()
            sc[_K - 1] = pltpu.async_copy(
                rows[(_K - 1) % 2], agg_sh.at[dst_v.at[_K - 1]],
                ssem[(_K - 1) % 2], add=True)
            sc[_K - 1].wait()
            return 0
        lax.fori_loop(0, nstage, s_body, 0)
        plsc.subcore_barrier()

        for k in range(n_wb):
            r0 = sid * rows_per_tile + k * _CH
            pltpu.sync_copy(agg_sh.at[pl.ds(r0, _CH)], rows_v)
            pltpu.sync_copy(rows_v, part_hbm.at[cid, pl.ds(r0, _CH)])

    return pl.kernel(
        body,
        out_type=jax.ShapeDtypeStruct((_NC, npad, d), jnp.float32),
        mesh=mesh,
        compiler_params=pltpu.CompilerParams(use_tc_tiling_on_sc=False),
        scratch_types=[
            pltpu.VMEM((_K, _CH), jnp.int32),
            pltpu.VMEM((_K, _CH), jnp.int32),
            pltpu.VMEM((_CH, d), jnp.float32),
            pltpu.VMEM((_CH, d), jnp.float32),
            pltpu.VMEM_SHARED((npad, d), jnp.float32),
            pltpu.SemaphoreType.DMA,
            pltpu.SemaphoreType.DMA,
            pltpu.SemaphoreType.DMA,
            pltpu.SemaphoreType.DMA,
        ],
    )


def kernel(x, edge_index, W0, b0, W1, b1, Wpg, bpg, Wpd, bpd, Wv, bv):
    n, d = x.shape
    e = edge_index.shape[1]

    nchunks = -(-(-(-e // (_NW * _CH))) // _K) * _K
    epw = nchunks * _CH
    e_pad = _NW * epw
    npad = -(-(n + 1) // (_NS * _CH)) * (_NS * _CH)

    pad = e_pad - e
    src = jnp.concatenate([edge_index[0], jnp.zeros((pad,), jnp.int32)])
    dst = jnp.concatenate([edge_index[1], jnp.full((pad,), n, jnp.int32)])
    src3 = src.reshape(_NW, nchunks, _CH)
    dst3 = dst.reshape(_NW, nchunks, _CH)
    srcf = src.reshape(_NW * nchunks, _CH)
    dstf = dst.reshape(_NW * nchunks, _CH)

    per_sid = 2 * nchunks
    nc0 = _NF * _K
    nc1 = per_sid - nc0

    deg_all = _make_deg(npad, nchunks)(dst3)

    p1 = _make_mp(n, npad, nc0, nc1, d)(_copy(x), srcf, dstf)
    h1 = _gcn_dense(p1, deg_all, W0, b0.reshape(1, d))
    p2 = _make_mp(npad, npad, nc0, nc1, d)(h1, srcf, dstf)
    pi, v = _heads(
        n, p2, deg_all, b1.reshape(1, d), W1,
        Wpg, Wpd, Wv,
        bpg.reshape(1, 1), bpd.reshape(1, 1), bv.reshape(1, 1),
    )
    return (pi, v)


# speedup vs baseline: 1.0339x; 1.0339x over previous
import functools

import jax
import jax.numpy as jnp
from jax import lax
from jax.experimental import pallas as pl
from jax.experimental.pallas import tpu as pltpu
from jax.experimental.pallas import tpu_sc as plsc

_NC = 2
_NS = 16
_NW = _NC * _NS
_CH = 128
_K = 16
_NF = 7


def _copy_body(x_ref, o_ref):
    o_ref[...] = x_ref[...]


def _copy(x, block_rows=1000):
    n, d = x.shape
    return pl.pallas_call(
        _copy_body,
        grid=(n // block_rows,),
        in_specs=[pl.BlockSpec((block_rows, d), lambda i: (i, 0))],
        out_specs=pl.BlockSpec((block_rows, d), lambda i: (i, 0)),
        out_shape=jax.ShapeDtypeStruct((n, d), jnp.float32),
    )(x)


def _gcn_body(p_ref, deg_ref, b_ref, w_ref, o_ref):
    agg = p_ref[0] + p_ref[1]
    deg = jnp.sum(deg_ref[...], axis=0)[:, None]
    agg = agg / jnp.maximum(deg, 1.0)
    o_ref[...] = jnp.maximum(
        jnp.dot(agg, w_ref[...], preferred_element_type=jnp.float32) + b_ref[...],
        0.0,
    )


def _gcn_dense(p, deg_all, w, b, block_rows=1024):
    npad, d = p.shape[1], p.shape[2]
    dout = w.shape[1]
    grid = (npad // block_rows,)
    return pl.pallas_call(
        _gcn_body,
        grid=grid,
        in_specs=[
            pl.BlockSpec((2, block_rows, d), lambda i: (0, i, 0)),
            pl.BlockSpec((_NW, block_rows), lambda i: (0, i)),
            pl.BlockSpec((1, dout), lambda i: (0, 0)),
            pl.BlockSpec((d, dout), lambda i: (0, 0)),
        ],
        out_specs=pl.BlockSpec((block_rows, dout), lambda i: (i, 0)),
        out_shape=jax.ShapeDtypeStruct((npad, dout), jnp.float32),
    )(p, deg_all, b, w)


def _make_heads_body(n):
    def _heads_body(q_ref, deg_ref, b1_ref, w1_ref, wpg_ref, wpd_ref, wv_ref,
                    bpg_ref, bpd_ref, bv_ref, pi_ref, v_ref):
        agg = q_ref[0, pl.ds(0, n), :] + q_ref[1, pl.ds(0, n), :]
        deg = jnp.sum(deg_ref[:, pl.ds(0, n)], axis=0)[:, None]
        agg = agg / jnp.maximum(deg, 1.0)
        h2 = jnp.maximum(
            jnp.dot(agg, w1_ref[...], preferred_element_type=jnp.float32) + b1_ref[...],
            0.0,
        )
        mn = jnp.mean(h2, axis=0, keepdims=True)
        pi_ref[pl.ds(0, n), :] = (
            jnp.dot(h2, wpg_ref[...], preferred_element_type=jnp.float32) + bpg_ref[...]
        )
        pi_ref[pl.ds(n, 1), :] = (
            jnp.dot(mn, wpd_ref[...], preferred_element_type=jnp.float32) + bpd_ref[...]
        )
        v_ref[...] = jnp.dot(mn, wv_ref[...], preferred_element_type=jnp.float32) + bv_ref[...]
    return _heads_body


def _heads(n, q, deg_all, b1, w1, wpg, wpd, wv, bpg, bpd, bv):
    return pl.pallas_call(
        _make_heads_body(n),
        out_shape=(
            jax.ShapeDtypeStruct((n + 1, 1), jnp.float32),
            jax.ShapeDtypeStruct((1, 1), jnp.float32),
        ),
    )(q, deg_all, b1, w1, wpg, wpd, wv, bpg, bpd, bv)


@functools.lru_cache(maxsize=None)
def _make_deg(npad, nchunks):
    nstage = nchunks // _K
    assert nchunks % _K == 0

    mesh = plsc.VectorSubcoreMesh(core_axis_name="c", subcore_axis_name="s")

    def body(dst_hbm, deg_hbm, dst_v, deg_l):
        cid = lax.axis_index("c")
        sid = lax.axis_index("s")
        wid = sid * _NC + cid

        zeros16 = jnp.zeros((16,), jnp.float32)
        ones16 = jnp.ones((16,), jnp.float32)

        def z_body(i, _):
            deg_l[pl.ds(i * 16, 16)] = zeros16
            return 0
        lax.fori_loop(0, npad // 16, z_body, 0)

        def s_body(s, _):
            pltpu.sync_copy(dst_hbm.at[wid, pl.ds(s * _K, _K)], dst_v)

            def c_body(j, _):
                for l in range(_CH // 16):
                    idxv = dst_v[j, pl.ds(l * 16, 16)]
                    plsc.addupdate_scatter(deg_l, [idxv], ones16)
                return 0
            lax.fori_loop(0, _K, c_body, 0)
            return 0
        lax.fori_loop(0, nstage, s_body, 0)

        pltpu.sync_copy(deg_l, deg_hbm.at[wid])

    return pl.kernel(
        body,
        out_type=jax.ShapeDtypeStruct((_NW, npad), jnp.float32),
        mesh=mesh,
        compiler_params=pltpu.CompilerParams(
            use_tc_tiling_on_sc=False, needs_layout_passes=False),
        scratch_types=[
            pltpu.VMEM((_K, _CH), jnp.int32),
            pltpu.VMEM((npad,), jnp.float32),
        ],
    )


@functools.lru_cache(maxsize=None)
def _make_mp(hwrows, npad, nchunks0, nchunks1, d):
    rows_per_tile = npad // _NS
    n_wb = rows_per_tile // _CH
    per_sid = nchunks0 + nchunks1
    assert rows_per_tile % _CH == 0
    assert nchunks0 % _K == 0 and nchunks1 % _K == 0

    mesh = plsc.VectorSubcoreMesh(core_axis_name="c", subcore_axis_name="s")

    def body(hw_hbm, src_hbm, dst_hbm, part_hbm,
             src_v, dst_v, rows_v, rows1_v, agg_sh, sem, g1, s0, s1):
        cid = lax.axis_index("c")
        sid = lax.axis_index("s")
        start = sid * per_sid + cid * nchunks0
        nstage = jnp.where(cid == 0, nchunks0 // _K, nchunks1 // _K)

        zeros16 = jnp.zeros((16,), jnp.float32)

        def z_body(i, _):
            def z_inner(k, _):
                rows_v[i, pl.ds(k * 16, 16)] = zeros16
                return 0
            lax.fori_loop(0, d // 16, z_inner, 0)
            return 0
        lax.fori_loop(0, _CH, z_body, 0)

        for k in range(n_wb):
            r0 = sid * rows_per_tile + k * _CH
            pltpu.sync_copy(rows_v, agg_sh.at[pl.ds(r0, _CH)])
        plsc.subcore_barrier()

        def s_body(st, _):
            r0 = start + st * _K
            pltpu.sync_copy(src_hbm.at[pl.ds(r0, _K)], src_v)
            pltpu.sync_copy(dst_hbm.at[pl.ds(r0, _K)], dst_v)

            rows = (rows_v, rows1_v)
            gsem = (sem, g1)
            ssem = (s0, s1)
            g = [None] * _K
            sc = [None] * _K
            g[0] = pltpu.async_copy(hw_hbm.at[src_v.at[0]], rows[0], gsem[0])
            for c in range(1, _K):
                b = c % 2
                g[c - 1].wait()
                if c >= 2:
                    sc[c - 2].wait()
                sc[c - 1] = pltpu.async_copy(
                    rows[1 - b], agg_sh.at[dst_v.at[c - 1]], ssem[1 - b], add=True)
                g[c] = pltpu.async_copy(hw_hbm.at[src_v.at[c]], rows[b], gsem[b])
            g[_K - 1].wait()
            sc[_K - 2].wait()
            sc[_K - 1] = pltpu.async_copy(
                rows[(_K - 1) % 2], agg_sh.at[dst_v.at[_K - 1]],
                ssem[(_K - 1) % 2], add=True)
            sc[_K - 1].wait()
            return 0
        lax.fori_loop(0, nstage, s_body, 0)
        plsc.subcore_barrier()

        for k in range(n_wb):
            r0 = sid * rows_per_tile + k * _CH
            pltpu.sync_copy(agg_sh.at[pl.ds(r0, _CH)], rows_v)
            pltpu.sync_copy(rows_v, part_hbm.at[cid, pl.ds(r0, _CH)])

    return pl.kernel(
        body,
        out_type=jax.ShapeDtypeStruct((_NC, npad, d), jnp.float32),
        mesh=mesh,
        compiler_params=pltpu.CompilerParams(use_tc_tiling_on_sc=False),
        scratch_types=[
            pltpu.VMEM((_K, _CH), jnp.int32),
            pltpu.VMEM((_K, _CH), jnp.int32),
            pltpu.VMEM((_CH, d), jnp.float32),
            pltpu.VMEM((_CH, d), jnp.float32),
            pltpu.VMEM_SHARED((npad, d), jnp.float32),
            pltpu.SemaphoreType.DMA,
            pltpu.SemaphoreType.DMA,
            pltpu.SemaphoreType.DMA,
            pltpu.SemaphoreType.DMA,
        ],
    )


def kernel(x, edge_index, W0, b0, W1, b1, Wpg, bpg, Wpd, bpd, Wv, bv):
    n, d = x.shape
    e = edge_index.shape[1]

    nchunks = -(-(-(-e // (_NW * _CH))) // _K) * _K
    epw = nchunks * _CH
    e_pad = _NW * epw
    npad = -(-(n + 1) // (_NS * _CH)) * (_NS * _CH)

    pad = e_pad - e
    src = jnp.concatenate([edge_index[0], jnp.zeros((pad,), jnp.int32)])
    dst = jnp.concatenate([edge_index[1], jnp.full((pad,), n, jnp.int32)])
    src3 = src.reshape(_NW, nchunks, _CH)
    dst3 = dst.reshape(_NW, nchunks, _CH)
    srcf = src.reshape(_NW * nchunks, _CH)
    dstf = dst.reshape(_NW * nchunks, _CH)

    per_sid = 2 * nchunks
    nc0 = _NF * _K
    nc1 = per_sid - nc0

    deg_all = _make_deg(npad, nchunks)(dst3)

    p1 = _make_mp(n, npad, nc0, nc1, d)(_copy(x), srcf, dstf)
    h1 = _gcn_dense(p1, deg_all, W0, b0.reshape(1, d))
    p2 = _make_mp(npad, npad, nc0, nc1, d)(h1, srcf, dstf)
    pi, v = _heads(
        n, p2, deg_all, b1.reshape(1, d), W1,
        Wpg, Wpd, Wv,
        bpg.reshape(1, 1), bpd.reshape(1, 1), bv.reshape(1, 1),
    )
    return (pi, v)
